# baseline (speedup 1.0000x reference)
import functools

import jax
import jax.numpy as jnp
from jax import lax
from jax.experimental import pallas as pl
from jax.experimental.pallas import tpu as pltpu
from jax.experimental.pallas import tpu_sc as plsc

N_NODES = 10000
N_EDGES = 320000
D = 128
D_EDGE = 16

NC = 2
NS = 16
NW = NC * NS

N_PAD = 10240
ROWS_PER_TILE = N_PAD // NS
VEC = 16

N_HALVES = 2
NE_H = N_EDGES // N_HALVES


def _sc_mesh():
    return plsc.VectorSubcoreMesh(core_axis_name="c", subcore_axis_name="s",
                                  num_cores=NC, num_subcores=NS)


@functools.cache
def _sc_gather(ne, gch):
    e_per_w = ne // NW
    n_chunks = e_per_w // gch
    assert e_per_w * NW == ne and n_chunks * gch == e_per_w
    assert n_chunks % 2 == 1 and gch % 8 == 0 and gch <= 128

    @functools.partial(
        pl.kernel,
        out_type=jax.ShapeDtypeStruct((ne, D), jnp.float32),
        mesh=_sc_mesh(),
        scratch_types=[
            pltpu.VMEM((e_per_w,), jnp.int32),
            pltpu.VMEM((e_per_w,), jnp.int32),
            pltpu.VMEM((2, gch, D), jnp.float32),
            pltpu.VMEM((2, gch, D), jnp.float32),
            pltpu.VMEM((2, gch, D), jnp.float32),
            pltpu.SemaphoreType.DMA,
            pltpu.SemaphoreType.DMA,
            pltpu.SemaphoreType.DMA,
            pltpu.SemaphoreType.DMA,
            pltpu.SemaphoreType.DMA,
            pltpu.SemaphoreType.DMA,
        ],
    )
    def body(pa_hbm, pb_hbm, src_hbm, dst_hbm, g_hbm,
             idxs_v, idxd_v, buf_a, buf_b, buf_o,
             sa0, sa1, sb0, sb1, sw0, sw1):
        c = lax.axis_index("c")
        s = lax.axis_index("s")
        wid = s * NC + c
        ebase = wid * e_per_w
        sem_a = (sa0, sa1)
        sem_b = (sb0, sb1)
        sem_w = (sw0, sw1)

        pltpu.sync_copy(src_hbm.at[pl.ds(ebase, e_per_w)], idxs_v)
        pltpu.sync_copy(dst_hbm.at[pl.ds(ebase, e_per_w)], idxd_v)

        def fire(j, b):
            pltpu.async_copy(pa_hbm.at[idxs_v.at[pl.ds(j * gch, gch)]],
                             buf_a.at[b], sem_a[b])
            pltpu.async_copy(pb_hbm.at[idxd_v.at[pl.ds(j * gch, gch)]],
                             buf_b.at[b], sem_b[b])

        def wait_gather(j, b):
            pltpu.make_async_copy(pa_hbm.at[idxs_v.at[pl.ds(j * gch, gch)]],
                                  buf_a.at[b], sem_a[b]).wait()
            pltpu.make_async_copy(pb_hbm.at[idxd_v.at[pl.ds(j * gch, gch)]],
                                  buf_b.at[b], sem_b[b]).wait()

        def drain_write(b):
            pltpu.make_async_copy(buf_o.at[b], g_hbm.at[pl.ds(ebase, gch)],
                                  sem_w[b]).wait()

        def add_and_write(j, b):
            @plsc.parallel_loop(0, gch, unroll=4)
            def _(r):
                for k in range(D // VEC):
                    sl = pl.ds(k * VEC, VEC)
                    buf_o[b, r, sl] = buf_a[b, r, sl] + buf_b[b, r, sl]
            pltpu.async_copy(buf_o.at[b],
                             g_hbm.at[pl.ds(ebase + j * gch, gch)], sem_w[b])

        fire(0, 0)
        fire(1, 1)

        @pl.loop(0, n_chunks // 2)
        def _(t):
            for b in range(2):
                j = 2 * t + b
                wait_gather(j, b)

                @pl.when(t > 0)
                def _():
                    drain_write(b)

                add_and_write(j, b)

                @pl.when(j + 2 < n_chunks)
                def _():
                    fire(j + 2, b)

        jl = n_chunks - 1
        wait_gather(jl, 0)
        drain_write(0)
        add_and_write(jl, 0)
        drain_write(0)
        drain_write(1)

    return body


@functools.cache
def _sc_scatter(ne, gch):
    e_per_w = ne // NW
    n_chunks = e_per_w // gch
    assert e_per_w * NW == ne and n_chunks * gch == e_per_w
    assert n_chunks % 3 == 2 and gch % 8 == 0 and gch <= 128

    @functools.partial(
        pl.kernel,
        out_type=jax.ShapeDtypeStruct((NC, N_PAD, D), jnp.float32),
        mesh=_sc_mesh(),
        scratch_types=[
            pltpu.VMEM((n_chunks, gch), jnp.int32),
            pltpu.VMEM((3, gch, D), jnp.float32),
            pltpu.VMEM_SHARED((N_PAD, D), jnp.float32),
            pltpu.SemaphoreType.DMA,
            pltpu.SemaphoreType.DMA,
            pltpu.SemaphoreType.DMA,
            pltpu.SemaphoreType.DMA,
            pltpu.SemaphoreType.DMA,
            pltpu.SemaphoreType.DMA,
        ],
    )
    def body(msg_hbm, dst3_hbm, zeros_hbm, part_hbm, idx_v, rows_v, acc,
             sl0, sl1, sl2, ss0, ss1, ss2):
        c = lax.axis_index("c")
        s = lax.axis_index("s")
        wid = s * NC + c
        ebase = wid * e_per_w
        rbase = s * ROWS_PER_TILE
        sem_l = (sl0, sl1, sl2)
        sem_s = (ss0, ss1, ss2)

        pltpu.sync_copy(zeros_hbm.at[pl.ds(rbase, ROWS_PER_TILE)],
                        acc.at[pl.ds(rbase, ROWS_PER_TILE)])
        pltpu.sync_copy(dst3_hbm.at[wid], idx_v)
        plsc.subcore_barrier()

        def fire_load(j, b):
            pltpu.async_copy(msg_hbm.at[pl.ds(ebase + j * gch, gch)],
                             rows_v.at[b], sem_l[b])

        def wait_load(j, b):
            pltpu.make_async_copy(msg_hbm.at[pl.ds(ebase + j * gch, gch)],
                                  rows_v.at[b], sem_l[b]).wait()

        def drain_scatter(b):
            pltpu.make_async_copy(rows_v.at[b], acc.at[idx_v.at[0]],
                                  sem_s[b]).wait()

        def visit(j, b, bp):
            wait_load(j, b)
            pltpu.async_copy(rows_v.at[b], acc.at[idx_v.at[j]], sem_s[b],
                             add=True)

            @pl.when(j >= 1)
            def _():
                drain_scatter(bp)

            @pl.when(j + 2 < n_chunks)
            def _():
                fire_load(j + 2, bp)

        fire_load(0, 0)
        fire_load(1, 1)

        @pl.loop(0, n_chunks // 3)
        def _(t):
            for b in range(3):
                visit(3 * t + b, b, (b + 2) % 3)

        visit(n_chunks - 2, 0, 2)
        visit(n_chunks - 1, 1, 0)
        drain_scatter(1)

        plsc.subcore_barrier()
        pltpu.sync_copy(acc.at[pl.ds(rbase, ROWS_PER_TILE)],
                        part_hbm.at[c, pl.ds(rbase, ROWS_PER_TILE)])

    return body


_NB = 2000
_EB = 2000

_full = lambda shape: pl.BlockSpec(shape, lambda i: (0,) * len(shape))


def _pre_body(x_ref, wa_ref, wb_ref, bin_ref, pa_ref, pb_ref):
    xb = x_ref[...]
    pa_ref[...] = jnp.dot(xb, wa_ref[...], preferred_element_type=jnp.float32)
    pb_ref[...] = (jnp.dot(xb, wb_ref[...], preferred_element_type=jnp.float32)
                   + bin_ref[...])


def _tc_pre(x, wa, wb, b_in):
    return pl.pallas_call(
        _pre_body,
        grid=(N_NODES // _NB,),
        in_specs=[
            pl.BlockSpec((_NB, D), lambda i: (i, 0)),
            _full((D, D)), _full((D, D)), _full((1, D)),
        ],
        out_specs=[
            pl.BlockSpec((_NB, D), lambda i: (i, 0)),
            pl.BlockSpec((_NB, D), lambda i: (i, 0)),
        ],
        out_shape=[
            jax.ShapeDtypeStruct((N_NODES, D), jnp.float32),
            jax.ShapeDtypeStruct((N_NODES, D), jnp.float32),
        ],
    )(x, wa, wb, b_in)


def _msg_body(g_ref, ea_ref, wc_ref, w1_ref, b1_ref, w2_ref, b2_ref,
              out_ref):
    bf = jnp.bfloat16
    h = g_ref[...] + jnp.dot(
        ea_ref[...], wc_ref[...], preferred_element_type=jnp.float32)
    h = jnp.maximum(h, 0.0)
    h = h + jnp.maximum(
        jnp.dot(h.astype(bf), w1_ref[...].astype(bf),
                preferred_element_type=jnp.float32)
        + b1_ref[...], 0.0)
    out_ref[...] = h + jnp.maximum(
        jnp.dot(h.astype(bf), w2_ref[...].astype(bf),
                preferred_element_type=jnp.float32)
        + b2_ref[...], 0.0)


def _tc_msg(g, ea, wc, w1, b1, w2, b2):
    ne = g.shape[0]
    return pl.pallas_call(
        _msg_body,
        grid=(ne // _EB,),
        in_specs=[
            pl.BlockSpec((_EB, D), lambda i: (i, 0)),
            pl.BlockSpec((_EB, D_EDGE), lambda i: (i, 0)),
            _full((D_EDGE, D)), _full((D, D)), _full((1, D)),
            _full((D, D)), _full((1, D)),
        ],
        out_specs=pl.BlockSpec((_EB, D), lambda i: (i, 0)),
        out_shape=jax.ShapeDtypeStruct((ne, D), jnp.float32),
    )(g, ea, wc, w1, b1, w2, b2)


def _upd_body(p0_ref, p1_ref, p2_ref, p3_ref, x_ref, wi_ref, bi_ref,
              w1_ref, b1_ref, w2_ref, b2_ref, out_ref):
    agg = (p0_ref[...] + p1_ref[...]) + (p2_ref[...] + p3_ref[...])
    h = jnp.maximum(
        jnp.dot(agg, wi_ref[...], preferred_element_type=jnp.float32)
        + bi_ref[...], 0.0)
    h = h + jnp.maximum(
        jnp.dot(h, w1_ref[...], preferred_element_type=jnp.float32)
        + b1_ref[...], 0.0)
    h = h + jnp.maximum(
        jnp.dot(h, w2_ref[...], preferred_element_type=jnp.float32)
        + b2_ref[...], 0.0)
    out_ref[...] = x_ref[...] + h


def _tc_upd(parts, x, wi, bi, w1, b1, w2, b2):
    nblk = pl.BlockSpec((_NB, D), lambda i: (i, 0))
    return pl.pallas_call(
        _upd_body,
        grid=(N_NODES // _NB,),
        in_specs=[
            nblk, nblk, nblk, nblk, nblk,
            _full((D, D)), _full((1, D)),
            _full((D, D)), _full((1, D)),
            _full((D, D)), _full((1, D)),
        ],
        out_specs=nblk,
        out_shape=jax.ShapeDtypeStruct((N_NODES, D), jnp.float32),
    )(*parts, x, wi, bi, w1, b1, w2, b2)


def kernel(x, edge_index, edge_attr,
           msg_Win, msg_bin, msg_W1, msg_b1, msg_W2, msg_b2,
           upd_Win, upd_bin, upd_W1, upd_b1, upd_W2, upd_b2):
    src = edge_index[0].astype(jnp.int32)
    dst = edge_index[1].astype(jnp.int32)
    wa = msg_Win[:D]
    wb = msg_Win[D:2 * D]
    wc = msg_Win[2 * D:]

    pa, pb = _tc_pre(x, wa, wb, msg_bin.reshape(1, D))
    zeros = jnp.zeros((N_PAD, D), jnp.float32)

    gch = 40
    n_chunks = NE_H // NW // gch
    parts = []
    for half in range(N_HALVES):
        lo, hi = half * NE_H, (half + 1) * NE_H
        src_h, dst_h = src[lo:hi], dst[lo:hi]
        g = _sc_gather(NE_H, gch)(pa, pb, src_h, dst_h)
        msg = _tc_msg(g, edge_attr[lo:hi], wc,
                      msg_W1, msg_b1.reshape(1, D),
                      msg_W2, msg_b2.reshape(1, D))
        dst3 = dst_h.reshape(NW, n_chunks, gch)
        part = _sc_scatter(NE_H, gch)(msg, dst3, zeros)
        parts.extend([part[0, :N_NODES], part[1, :N_NODES]])

    out = _tc_upd(parts, x,
                  upd_Win, upd_bin.reshape(1, D),
                  upd_W1, upd_b1.reshape(1, D), upd_W2, upd_b2.reshape(1, D))
    return out


# speedup vs baseline: 1.0413x; 1.0413x over previous
import functools

import jax
import jax.numpy as jnp
from jax import lax
from jax.experimental import pallas as pl
from jax.experimental.pallas import tpu as pltpu
from jax.experimental.pallas import tpu_sc as plsc

N_NODES = 10000
N_EDGES = 320000
D = 128
D_EDGE = 16

NC = 2
NS = 16
NW = NC * NS

N_PAD = 10240
ROWS_PER_TILE = N_PAD // NS
VEC = 16

N_HALVES = 2
NE_H = N_EDGES // N_HALVES


def _sc_mesh():
    return plsc.VectorSubcoreMesh(core_axis_name="c", subcore_axis_name="s",
                                  num_cores=NC, num_subcores=NS)


@functools.cache
def _sc_gather(ne, gch):
    e_per_w = ne // NW
    n_chunks = e_per_w // gch
    assert e_per_w * NW == ne and n_chunks * gch == e_per_w
    assert n_chunks >= 4 and gch % 8 == 0 and gch <= 128

    @functools.partial(
        pl.kernel,
        out_type=jax.ShapeDtypeStruct((ne, D), jnp.float32),
        mesh=_sc_mesh(),
        scratch_types=[
            pltpu.VMEM((e_per_w,), jnp.int32),
            pltpu.VMEM((e_per_w,), jnp.int32),
            pltpu.VMEM((2, gch, D), jnp.float32),
            pltpu.VMEM((2, gch, D), jnp.float32),
            pltpu.VMEM((2, gch, D), jnp.float32),
            pltpu.SemaphoreType.DMA,
            pltpu.SemaphoreType.DMA,
            pltpu.SemaphoreType.DMA,
            pltpu.SemaphoreType.DMA,
            pltpu.SemaphoreType.DMA,
            pltpu.SemaphoreType.DMA,
        ],
    )
    def body(pa_hbm, pb_hbm, src_hbm, dst_hbm, g_hbm,
             idxs_v, idxd_v, buf_a, buf_b, buf_o,
             sa0, sa1, sb0, sb1, sw0, sw1):
        c = lax.axis_index("c")
        s = lax.axis_index("s")
        wid = s * NC + c
        ebase = wid * e_per_w
        sem_a = (sa0, sa1)
        sem_b = (sb0, sb1)
        sem_w = (sw0, sw1)

        pltpu.sync_copy(src_hbm.at[pl.ds(ebase, e_per_w)], idxs_v)
        pltpu.sync_copy(dst_hbm.at[pl.ds(ebase, e_per_w)], idxd_v)

        def fire(j, b):
            pltpu.async_copy(pa_hbm.at[idxs_v.at[pl.ds(j * gch, gch)]],
                             buf_a.at[b], sem_a[b])
            pltpu.async_copy(pb_hbm.at[idxd_v.at[pl.ds(j * gch, gch)]],
                             buf_b.at[b], sem_b[b])

        def wait_gather(j, b):
            pltpu.make_async_copy(pa_hbm.at[idxs_v.at[pl.ds(j * gch, gch)]],
                                  buf_a.at[b], sem_a[b]).wait()
            pltpu.make_async_copy(pb_hbm.at[idxd_v.at[pl.ds(j * gch, gch)]],
                                  buf_b.at[b], sem_b[b]).wait()

        def drain_write(b):
            pltpu.make_async_copy(buf_o.at[b], g_hbm.at[pl.ds(ebase, gch)],
                                  sem_w[b]).wait()

        def add_and_write(j, b):
            @plsc.parallel_loop(0, gch, unroll=4)
            def _(r):
                for k in range(D // VEC):
                    sl = pl.ds(k * VEC, VEC)
                    buf_o[b, r, sl] = buf_a[b, r, sl] + buf_b[b, r, sl]
            pltpu.async_copy(buf_o.at[b],
                             g_hbm.at[pl.ds(ebase + j * gch, gch)], sem_w[b])

        fire(0, 0)
        fire(1, 1)

        @pl.loop(0, n_chunks // 2)
        def _(t):
            for b in range(2):
                j = 2 * t + b
                wait_gather(j, b)

                @pl.when(t > 0)
                def _():
                    drain_write(b)

                add_and_write(j, b)

                @pl.when(j + 2 < n_chunks)
                def _():
                    fire(j + 2, b)

        if n_chunks % 2:
            jl = n_chunks - 1
            wait_gather(jl, 0)
            drain_write(0)
            add_and_write(jl, 0)
        drain_write(0)
        drain_write(1)

    return body


def _scatter_ring(msg_hbm, idx_v, rows_v, sem_l, sem_s, acc,
                  ebase, gch, n_chunks):
    def fire_load(j, b):
        pltpu.async_copy(msg_hbm.at[pl.ds(ebase + j * gch, gch)],
                         rows_v.at[b], sem_l[b])

    def wait_load(j, b):
        pltpu.make_async_copy(msg_hbm.at[pl.ds(ebase + j * gch, gch)],
                              rows_v.at[b], sem_l[b]).wait()

    def drain_scatter(b):
        pltpu.make_async_copy(rows_v.at[b], acc.at[idx_v.at[0]],
                              sem_s[b]).wait()

    def visit(j, b, bp):
        wait_load(j, b)
        pltpu.async_copy(rows_v.at[b], acc.at[idx_v.at[j]], sem_s[b],
                         add=True)

        @pl.when(j >= 1)
        def _():
            drain_scatter(bp)

        @pl.when(j + 2 < n_chunks)
        def _():
            fire_load(j + 2, bp)

    fire_load(0, 0)
    fire_load(1, 1)

    @pl.loop(0, n_chunks // 3)
    def _(t):
        for b in range(3):
            visit(3 * t + b, b, (b + 2) % 3)

    rem = n_chunks % 3
    for i in range(rem):
        j = n_chunks - rem + i
        visit(j, j % 3, (j + 2) % 3)
    drain_scatter((n_chunks - 1) % 3)


@functools.cache
def _sc_scatter(ne_a, gch_a, ne_b, gch_b):
    ew_a, ew_b = ne_a // NW, ne_b // NW
    nch_a, nch_b = ew_a // gch_a, ew_b // gch_b

    assert gch_a == gch_b

    @functools.partial(
        pl.kernel,
        out_type=jax.ShapeDtypeStruct((NC, N_PAD, D), jnp.float32),
        mesh=_sc_mesh(),
        scratch_types=[
            pltpu.VMEM((nch_a, gch_a), jnp.int32),
            pltpu.VMEM((nch_b, gch_b), jnp.int32),
            pltpu.VMEM((3, gch_a, D), jnp.float32),
            pltpu.VMEM_SHARED((N_PAD, D), jnp.float32),
            pltpu.SemaphoreType.DMA,
            pltpu.SemaphoreType.DMA,
            pltpu.SemaphoreType.DMA,
            pltpu.SemaphoreType.DMA,
            pltpu.SemaphoreType.DMA,
            pltpu.SemaphoreType.DMA,
        ],
    )
    def body(msga_hbm, msgb_hbm, dsta_hbm, dstb_hbm, zeros_hbm, part_hbm,
             idxa_v, idxb_v, rows_v, acc,
             sl0, sl1, sl2, ss0, ss1, ss2):
        c = lax.axis_index("c")
        s = lax.axis_index("s")
        wid = s * NC + c
        rbase = s * ROWS_PER_TILE
        sem_l = (sl0, sl1, sl2)
        sem_s = (ss0, ss1, ss2)

        pltpu.sync_copy(zeros_hbm.at[pl.ds(rbase, ROWS_PER_TILE)],
                        acc.at[pl.ds(rbase, ROWS_PER_TILE)])
        pltpu.sync_copy(dsta_hbm.at[wid], idxa_v)
        pltpu.sync_copy(dstb_hbm.at[wid], idxb_v)
        plsc.subcore_barrier()

        _scatter_ring(msga_hbm, idxa_v, rows_v, sem_l, sem_s, acc,
                      wid * ew_a, gch_a, nch_a)
        _scatter_ring(msgb_hbm, idxb_v, rows_v, sem_l, sem_s, acc,
                      wid * ew_b, gch_b, nch_b)

        plsc.subcore_barrier()
        pltpu.sync_copy(acc.at[pl.ds(rbase, ROWS_PER_TILE)],
                        part_hbm.at[c, pl.ds(rbase, ROWS_PER_TILE)])

    return body


_NB = 2000
_EB = 2560

_full = lambda shape: pl.BlockSpec(shape, lambda i: (0,) * len(shape))


def _pre_body(x_ref, wa_ref, wb_ref, bin_ref, pa_ref, pb_ref):
    xb = x_ref[...]
    pa_ref[...] = jnp.dot(xb, wa_ref[...], preferred_element_type=jnp.float32)
    pb_ref[...] = (jnp.dot(xb, wb_ref[...], preferred_element_type=jnp.float32)
                   + bin_ref[...])


def _tc_pre(x, wa, wb, b_in):
    return pl.pallas_call(
        _pre_body,
        grid=(N_NODES // _NB,),
        in_specs=[
            pl.BlockSpec((_NB, D), lambda i: (i, 0)),
            _full((D, D)), _full((D, D)), _full((1, D)),
        ],
        out_specs=[
            pl.BlockSpec((_NB, D), lambda i: (i, 0)),
            pl.BlockSpec((_NB, D), lambda i: (i, 0)),
        ],
        out_shape=[
            jax.ShapeDtypeStruct((N_NODES, D), jnp.float32),
            jax.ShapeDtypeStruct((N_NODES, D), jnp.float32),
        ],
    )(x, wa, wb, b_in)


def _msg_body(g_ref, ea_ref, wc_ref, w1_ref, b1_ref, w2_ref, b2_ref,
              out_ref):
    bf = jnp.bfloat16
    h = g_ref[...] + jnp.dot(
        ea_ref[...], wc_ref[...], preferred_element_type=jnp.float32)
    h = jnp.maximum(h, 0.0)
    h = h + jnp.maximum(
        jnp.dot(h.astype(bf), w1_ref[...].astype(bf),
                preferred_element_type=jnp.float32)
        + b1_ref[...], 0.0)
    out_ref[...] = h + jnp.maximum(
        jnp.dot(h.astype(bf), w2_ref[...].astype(bf),
                preferred_element_type=jnp.float32)
        + b2_ref[...], 0.0)


def _tc_msg(g, ea, wc, w1, b1, w2, b2):
    ne = g.shape[0]
    eb = next(e for e in range(_EB, 0, -8) if ne % e == 0)
    return pl.pallas_call(
        _msg_body,
        grid=(ne // eb,),
        in_specs=[
            pl.BlockSpec((eb, D), lambda i: (i, 0)),
            pl.BlockSpec((eb, D_EDGE), lambda i: (i, 0)),
            _full((D_EDGE, D)), _full((D, D)), _full((1, D)),
            _full((D, D)), _full((1, D)),
        ],
        out_specs=pl.BlockSpec((eb, D), lambda i: (i, 0)),
        out_shape=jax.ShapeDtypeStruct((ne, D), jnp.float32),
    )(g, ea, wc, w1, b1, w2, b2)


def _upd_body(p0_ref, p1_ref, x_ref, wi_ref, bi_ref,
              w1_ref, b1_ref, w2_ref, b2_ref, out_ref):
    agg = p0_ref[...] + p1_ref[...]
    h = jnp.maximum(
        jnp.dot(agg, wi_ref[...], preferred_element_type=jnp.float32)
        + bi_ref[...], 0.0)
    h = h + jnp.maximum(
        jnp.dot(h, w1_ref[...], preferred_element_type=jnp.float32)
        + b1_ref[...], 0.0)
    h = h + jnp.maximum(
        jnp.dot(h, w2_ref[...], preferred_element_type=jnp.float32)
        + b2_ref[...], 0.0)
    out_ref[...] = x_ref[...] + h


def _tc_upd(parts, x, wi, bi, w1, b1, w2, b2):
    nblk = pl.BlockSpec((_NB, D), lambda i: (i, 0))
    return pl.pallas_call(
        _upd_body,
        grid=(N_NODES // _NB,),
        in_specs=[
            nblk, nblk, nblk,
            _full((D, D)), _full((1, D)),
            _full((D, D)), _full((1, D)),
            _full((D, D)), _full((1, D)),
        ],
        out_specs=nblk,
        out_shape=jax.ShapeDtypeStruct((N_NODES, D), jnp.float32),
    )(*parts, x, wi, bi, w1, b1, w2, b2)


def kernel(x, edge_index, edge_attr,
           msg_Win, msg_bin, msg_W1, msg_b1, msg_W2, msg_b2,
           upd_Win, upd_bin, upd_W1, upd_b1, upd_W2, upd_b2):
    src = edge_index[0].astype(jnp.int32)
    dst = edge_index[1].astype(jnp.int32)
    wa = msg_Win[:D]
    wb = msg_Win[D:2 * D]
    wc = msg_Win[2 * D:]

    pa, pb = _tc_pre(x, wa, wb, msg_bin.reshape(1, D))
    zeros = jnp.zeros((N_PAD, D), jnp.float32)

    splits = ((0, 204800, 128), (204800, 115200, 120))
    scatter_gch = 80
    msgs, dst3s = [], []
    for lo, ne, gch in splits:
        hi = lo + ne
        src_h, dst_h = src[lo:hi], dst[lo:hi]
        g = _sc_gather(ne, gch)(pa, pb, src_h, dst_h)
        msg = _tc_msg(g, edge_attr[lo:hi], wc,
                      msg_W1, msg_b1.reshape(1, D),
                      msg_W2, msg_b2.reshape(1, D))
        msgs.append(msg)
        dst3s.append(dst_h.reshape(NW, ne // NW // scatter_gch, scatter_gch))

    (_, ne_a, _), (_, ne_b, _) = splits
    part = _sc_scatter(ne_a, scatter_gch, ne_b, scatter_gch)(
        msgs[0], msgs[1], dst3s[0], dst3s[1], zeros)
    parts = [part[0, :N_NODES], part[1, :N_NODES]]

    out = _tc_upd(parts, x,
                  upd_Win, upd_bin.reshape(1, D),
                  upd_W1, upd_b1.reshape(1, D), upd_W2, upd_b2.reshape(1, D))
    return out


# speedup vs baseline: 1.0670x; 1.0247x over previous
import functools

import jax
import jax.numpy as jnp
from jax import lax
from jax.experimental import pallas as pl
from jax.experimental.pallas import tpu as pltpu
from jax.experimental.pallas import tpu_sc as plsc

N_NODES = 10000
N_EDGES = 320000
D = 128
D_EDGE = 16

NC = 2
NS = 16
NW = NC * NS

N_PAD = 10240
ROWS_PER_TILE = N_PAD // NS
VEC = 16

N_HALVES = 2
NE_H = N_EDGES // N_HALVES


def _sc_mesh():
    return plsc.VectorSubcoreMesh(core_axis_name="c", subcore_axis_name="s",
                                  num_cores=NC, num_subcores=NS)


@functools.cache
def _sc_gather(ne, gch, lo):
    e_per_w = ne // NW
    n_chunks = e_per_w // gch
    assert e_per_w * NW == ne and n_chunks * gch == e_per_w
    assert n_chunks >= 4 and gch % 8 == 0 and gch <= 128

    @functools.partial(
        pl.kernel,
        out_type=jax.ShapeDtypeStruct((ne, D), jnp.float32),
        mesh=_sc_mesh(),
        scratch_types=[
            pltpu.VMEM((e_per_w,), jnp.int32),
            pltpu.VMEM((e_per_w,), jnp.int32),
            pltpu.VMEM((2, gch, D), jnp.float32),
            pltpu.VMEM((2, gch, D), jnp.float32),
            pltpu.VMEM((2, gch, D), jnp.float32),
            pltpu.SemaphoreType.DMA,
            pltpu.SemaphoreType.DMA,
            pltpu.SemaphoreType.DMA,
            pltpu.SemaphoreType.DMA,
            pltpu.SemaphoreType.DMA,
            pltpu.SemaphoreType.DMA,
        ],
    )
    def body(pa_hbm, pb_hbm, src_hbm, dst_hbm, g_hbm,
             idxs_v, idxd_v, buf_a, buf_b, buf_o,
             sa0, sa1, sb0, sb1, sw0, sw1):
        c = lax.axis_index("c")
        s = lax.axis_index("s")
        wid = s * NC + c
        ebase = wid * e_per_w
        sem_a = (sa0, sa1)
        sem_b = (sb0, sb1)
        sem_w = (sw0, sw1)

        pltpu.sync_copy(src_hbm.at[pl.ds(lo + ebase, e_per_w)], idxs_v)
        pltpu.sync_copy(dst_hbm.at[pl.ds(lo + ebase, e_per_w)], idxd_v)

        def fire(j, b):
            pltpu.async_copy(pa_hbm.at[idxs_v.at[pl.ds(j * gch, gch)]],
                             buf_a.at[b], sem_a[b])
            pltpu.async_copy(pb_hbm.at[idxd_v.at[pl.ds(j * gch, gch)]],
                             buf_b.at[b], sem_b[b])

        def wait_gather(j, b):
            pltpu.make_async_copy(pa_hbm.at[idxs_v.at[pl.ds(j * gch, gch)]],
                                  buf_a.at[b], sem_a[b]).wait()
            pltpu.make_async_copy(pb_hbm.at[idxd_v.at[pl.ds(j * gch, gch)]],
                                  buf_b.at[b], sem_b[b]).wait()

        def drain_write(b):
            pltpu.make_async_copy(buf_o.at[b], g_hbm.at[pl.ds(ebase, gch)],
                                  sem_w[b]).wait()

        def add_and_write(j, b):
            @plsc.parallel_loop(0, gch, unroll=4)
            def _(r):
                for k in range(D // VEC):
                    sl = pl.ds(k * VEC, VEC)
                    buf_o[b, r, sl] = buf_a[b, r, sl] + buf_b[b, r, sl]
            pltpu.async_copy(buf_o.at[b],
                             g_hbm.at[pl.ds(ebase + j * gch, gch)], sem_w[b])

        fire(0, 0)
        fire(1, 1)

        @pl.loop(0, n_chunks // 2)
        def _(t):
            for b in range(2):
                j = 2 * t + b
                wait_gather(j, b)

                @pl.when(t > 0)
                def _():
                    drain_write(b)

                add_and_write(j, b)

                @pl.when(j + 2 < n_chunks)
                def _():
                    fire(j + 2, b)

        if n_chunks % 2:
            jl = n_chunks - 1
            wait_gather(jl, 0)
            drain_write(0)
            add_and_write(jl, 0)
        drain_write(0)
        drain_write(1)

    return body


def _scatter_ring(msg_hbm, idx_v, rows_v, sem_l, sem_s, acc,
                  ebase, gch, n_chunks):
    def fire_load(j, b):
        pltpu.async_copy(msg_hbm.at[pl.ds(ebase + j * gch, gch)],
                         rows_v.at[b], sem_l[b])

    def wait_load(j, b):
        pltpu.make_async_copy(msg_hbm.at[pl.ds(ebase + j * gch, gch)],
                              rows_v.at[b], sem_l[b]).wait()

    def drain_scatter(b):
        pltpu.make_async_copy(rows_v.at[b], acc.at[idx_v.at[0]],
                              sem_s[b]).wait()

    def visit(j, b, bp):
        wait_load(j, b)
        pltpu.async_copy(rows_v.at[b], acc.at[idx_v.at[j]], sem_s[b],
                         add=True)

        @pl.when(j >= 1)
        def _():
            drain_scatter(bp)

        @pl.when(j + 2 < n_chunks)
        def _():
            fire_load(j + 2, bp)

    fire_load(0, 0)
    fire_load(1, 1)

    @pl.loop(0, n_chunks // 3)
    def _(t):
        for b in range(3):
            visit(3 * t + b, b, (b + 2) % 3)

    rem = n_chunks % 3
    for i in range(rem):
        j = n_chunks - rem + i
        visit(j, j % 3, (j + 2) % 3)
    drain_scatter((n_chunks - 1) % 3)


@functools.cache
def _sc_scatter(ne_a, gch_a, ne_b, gch_b):
    ew_a, ew_b = ne_a // NW, ne_b // NW
    nch_a, nch_b = ew_a // gch_a, ew_b // gch_b

    assert gch_a == gch_b

    @functools.partial(
        pl.kernel,
        out_type=jax.ShapeDtypeStruct((NC, N_PAD, D), jnp.float32),
        mesh=_sc_mesh(),
        scratch_types=[
            pltpu.VMEM((nch_a, gch_a), jnp.int32),
            pltpu.VMEM((nch_b, gch_b), jnp.int32),
            pltpu.VMEM((3, gch_a, D), jnp.float32),
            pltpu.VMEM_SHARED((N_PAD, D), jnp.float32),
            pltpu.SemaphoreType.DMA,
            pltpu.SemaphoreType.DMA,
            pltpu.SemaphoreType.DMA,
            pltpu.SemaphoreType.DMA,
            pltpu.SemaphoreType.DMA,
            pltpu.SemaphoreType.DMA,
        ],
    )
    def body(msga_hbm, msgb_hbm, dsta_hbm, dstb_hbm, zeros_hbm, part_hbm,
             idxa_v, idxb_v, rows_v, acc,
             sl0, sl1, sl2, ss0, ss1, ss2):
        c = lax.axis_index("c")
        s = lax.axis_index("s")
        wid = s * NC + c
        rbase = s * ROWS_PER_TILE
        sem_l = (sl0, sl1, sl2)
        sem_s = (ss0, ss1, ss2)

        pltpu.sync_copy(zeros_hbm.at[pl.ds(rbase, ROWS_PER_TILE)],
                        acc.at[pl.ds(rbase, ROWS_PER_TILE)])
        pltpu.sync_copy(dsta_hbm.at[wid], idxa_v)
        pltpu.sync_copy(dstb_hbm.at[wid], idxb_v)
        plsc.subcore_barrier()

        _scatter_ring(msga_hbm, idxa_v, rows_v, sem_l, sem_s, acc,
                      wid * ew_a, gch_a, nch_a)
        _scatter_ring(msgb_hbm, idxb_v, rows_v, sem_l, sem_s, acc,
                      wid * ew_b, gch_b, nch_b)

        plsc.subcore_barrier()
        pltpu.sync_copy(acc.at[pl.ds(rbase, ROWS_PER_TILE)],
                        part_hbm.at[c, pl.ds(rbase, ROWS_PER_TILE)])

    return body


_NB = 2000
_EB = 2560

_full = lambda shape: pl.BlockSpec(shape, lambda i: (0,) * len(shape))


def _pre_body(x_ref, wa_ref, wb_ref, bin_ref, pa_ref, pb_ref):
    xb = x_ref[...]
    pa_ref[...] = jnp.dot(xb, wa_ref[...], preferred_element_type=jnp.float32)
    pb_ref[...] = (jnp.dot(xb, wb_ref[...], preferred_element_type=jnp.float32)
                   + bin_ref[...])


def _tc_pre(x, wa, wb, b_in):
    return pl.pallas_call(
        _pre_body,
        grid=(N_NODES // _NB,),
        in_specs=[
            pl.BlockSpec((_NB, D), lambda i: (i, 0)),
            _full((D, D)), _full((D, D)), _full((1, D)),
        ],
        out_specs=[
            pl.BlockSpec((_NB, D), lambda i: (i, 0)),
            pl.BlockSpec((_NB, D), lambda i: (i, 0)),
        ],
        out_shape=[
            jax.ShapeDtypeStruct((N_NODES, D), jnp.float32),
            jax.ShapeDtypeStruct((N_NODES, D), jnp.float32),
        ],
    )(x, wa, wb, b_in)


def _msg_body(g_ref, ea_ref, wc_ref, w1_ref, b1_ref, w2_ref, b2_ref,
              out_ref):
    bf = jnp.bfloat16
    h = g_ref[...] + jnp.dot(
        ea_ref[...], wc_ref[...], preferred_element_type=jnp.float32)
    h = jnp.maximum(h, 0.0)
    h = h + jnp.maximum(
        jnp.dot(h.astype(bf), w1_ref[...].astype(bf),
                preferred_element_type=jnp.float32)
        + b1_ref[...], 0.0)
    out_ref[...] = h + jnp.maximum(
        jnp.dot(h.astype(bf), w2_ref[...].astype(bf),
                preferred_element_type=jnp.float32)
        + b2_ref[...], 0.0)


def _tc_msg(g, ea, wc, w1, b1, w2, b2, lo):
    ne = g.shape[0]
    eb = next(e for e in range(_EB, 0, -8) if ne % e == 0 and lo % e == 0)
    off = lo // eb
    return pl.pallas_call(
        _msg_body,
        grid=(ne // eb,),
        in_specs=[
            pl.BlockSpec((eb, D), lambda i: (i, 0)),
            pl.BlockSpec((eb, D_EDGE), lambda i: (i + off, 0)),
            _full((D_EDGE, D)), _full((D, D)), _full((1, D)),
            _full((D, D)), _full((1, D)),
        ],
        out_specs=pl.BlockSpec((eb, D), lambda i: (i, 0)),
        out_shape=jax.ShapeDtypeStruct((ne, D), jnp.float32),
    )(g, ea, wc, w1, b1, w2, b2)


def _upd_body(p0_ref, p1_ref, x_ref, wi_ref, bi_ref,
              w1_ref, b1_ref, w2_ref, b2_ref, out_ref):
    agg = p0_ref[...] + p1_ref[...]
    h = jnp.maximum(
        jnp.dot(agg, wi_ref[...], preferred_element_type=jnp.float32)
        + bi_ref[...], 0.0)
    h = h + jnp.maximum(
        jnp.dot(h, w1_ref[...], preferred_element_type=jnp.float32)
        + b1_ref[...], 0.0)
    h = h + jnp.maximum(
        jnp.dot(h, w2_ref[...], preferred_element_type=jnp.float32)
        + b2_ref[...], 0.0)
    out_ref[...] = x_ref[...] + h


def _tc_upd(parts, x, wi, bi, w1, b1, w2, b2):
    nblk = pl.BlockSpec((_NB, D), lambda i: (i, 0))
    return pl.pallas_call(
        _upd_body,
        grid=(N_NODES // _NB,),
        in_specs=[
            nblk, nblk, nblk,
            _full((D, D)), _full((1, D)),
            _full((D, D)), _full((1, D)),
            _full((D, D)), _full((1, D)),
        ],
        out_specs=nblk,
        out_shape=jax.ShapeDtypeStruct((N_NODES, D), jnp.float32),
    )(*parts, x, wi, bi, w1, b1, w2, b2)


def kernel(x, edge_index, edge_attr,
           msg_Win, msg_bin, msg_W1, msg_b1, msg_W2, msg_b2,
           upd_Win, upd_bin, upd_W1, upd_b1, upd_W2, upd_b2):
    src = edge_index[0].astype(jnp.int32)
    dst = edge_index[1].astype(jnp.int32)
    wa = msg_Win[:D]
    wb = msg_Win[D:2 * D]
    wc = msg_Win[2 * D:]

    pa, pb = _tc_pre(x, wa, wb, msg_bin.reshape(1, D))
    zeros = jnp.zeros((N_PAD, D), jnp.float32)

    splits = ((0, 204800, 128), (204800, 115200, 120))
    scatter_gch = 80
    msgs, dst3s = [], []
    for lo, ne, gch in splits:
        g = _sc_gather(ne, gch, lo)(pa, pb, src, dst)
        msg = _tc_msg(g, edge_attr, wc,
                      msg_W1, msg_b1.reshape(1, D),
                      msg_W2, msg_b2.reshape(1, D), lo)
        msgs.append(msg)
        dst3s.append(dst[lo:lo + ne]
                     .reshape(NW, ne // NW // scatter_gch, scatter_gch))

    (_, ne_a, _), (_, ne_b, _) = splits
    part = _sc_scatter(ne_a, scatter_gch, ne_b, scatter_gch)(
        msgs[0], msgs[1], dst3s[0], dst3s[1], zeros)
    parts = [part[0, :N_NODES], part[1, :N_NODES]]

    out = _tc_upd(parts, x,
                  upd_Win, upd_bin.reshape(1, D),
                  upd_W1, upd_b1.reshape(1, D), upd_W2, upd_b2.reshape(1, D))
    return out


# speedup vs baseline: 1.1899x; 1.1152x over previous
import functools

import jax
import jax.numpy as jnp
from jax import lax
from jax.experimental import pallas as pl
from jax.experimental.pallas import tpu as pltpu
from jax.experimental.pallas import tpu_sc as plsc

N_NODES = 10000
N_EDGES = 320000
D = 128
D_EDGE = 16

NC = 2
NS = 16
NW = NC * NS

N_PAD = 10240
ROWS_PER_TILE = N_PAD // NS
VEC = 16

N_HALVES = 2
NE_H = N_EDGES // N_HALVES


def _sc_mesh():
    return plsc.VectorSubcoreMesh(core_axis_name="c", subcore_axis_name="s",
                                  num_cores=NC, num_subcores=NS)


@functools.cache
def _sc_gather(ne, gch, lo):
    e_per_w = ne // NW
    n_chunks = e_per_w // gch
    assert e_per_w * NW == ne and n_chunks * gch == e_per_w
    assert n_chunks >= 4 and gch % 8 == 0 and gch <= 128

    @functools.partial(
        pl.kernel,
        out_type=jax.ShapeDtypeStruct((ne, D), jnp.float32),
        mesh=_sc_mesh(),
        scratch_types=[
            pltpu.VMEM((e_per_w,), jnp.int32),
            pltpu.VMEM((e_per_w,), jnp.int32),
            pltpu.VMEM((2, gch, D), jnp.float32),
            pltpu.VMEM((2, gch, D), jnp.float32),
            pltpu.VMEM((2, gch, D), jnp.float32),
            pltpu.SemaphoreType.DMA,
            pltpu.SemaphoreType.DMA,
            pltpu.SemaphoreType.DMA,
            pltpu.SemaphoreType.DMA,
            pltpu.SemaphoreType.DMA,
            pltpu.SemaphoreType.DMA,
        ],
    )
    def body(pa_hbm, pb_hbm, src_hbm, dst_hbm, g_hbm,
             idxs_v, idxd_v, buf_a, buf_b, buf_o,
             sa0, sa1, sb0, sb1, sw0, sw1):
        c = lax.axis_index("c")
        s = lax.axis_index("s")
        wid = s * NC + c
        ebase = wid * e_per_w
        sem_a = (sa0, sa1)
        sem_b = (sb0, sb1)
        sem_w = (sw0, sw1)

        pltpu.sync_copy(src_hbm.at[pl.ds(lo + ebase, e_per_w)], idxs_v)
        pltpu.sync_copy(dst_hbm.at[pl.ds(lo + ebase, e_per_w)], idxd_v)

        def fire(j, b):
            pltpu.async_copy(pa_hbm.at[idxs_v.at[pl.ds(j * gch, gch)]],
                             buf_a.at[b], sem_a[b])
            pltpu.async_copy(pb_hbm.at[idxd_v.at[pl.ds(j * gch, gch)]],
                             buf_b.at[b], sem_b[b])

        def wait_gather(j, b):
            pltpu.make_async_copy(pa_hbm.at[idxs_v.at[pl.ds(j * gch, gch)]],
                                  buf_a.at[b], sem_a[b]).wait()
            pltpu.make_async_copy(pb_hbm.at[idxd_v.at[pl.ds(j * gch, gch)]],
                                  buf_b.at[b], sem_b[b]).wait()

        def drain_write(b):
            pltpu.make_async_copy(buf_o.at[b], g_hbm.at[pl.ds(ebase, gch)],
                                  sem_w[b]).wait()

        def add_and_write(j, b):
            @plsc.parallel_loop(0, gch, unroll=4)
            def _(r):
                for k in range(D // VEC):
                    sl = pl.ds(k * VEC, VEC)
                    buf_o[b, r, sl] = buf_a[b, r, sl] + buf_b[b, r, sl]
            pltpu.async_copy(buf_o.at[b],
                             g_hbm.at[pl.ds(ebase + j * gch, gch)], sem_w[b])

        fire(0, 0)
        fire(1, 1)

        @pl.loop(0, n_chunks // 2)
        def _(t):
            for b in range(2):
                j = 2 * t + b
                wait_gather(j, b)

                @pl.when(t > 0)
                def _():
                    drain_write(b)

                add_and_write(j, b)

                @pl.when(j + 2 < n_chunks)
                def _():
                    fire(j + 2, b)

        if n_chunks % 2:
            jl = n_chunks - 1
            wait_gather(jl, 0)
            drain_write(0)
            add_and_write(jl, 0)
        drain_write(0)
        drain_write(1)

    return body


def _scatter_ring(msg_hbm, idx_v, rows_v, sem_l, sem_s, acc,
                  ebase, gch, n_chunks):
    def fire_load(j, b):
        pltpu.async_copy(msg_hbm.at[pl.ds(ebase + j * gch, gch)],
                         rows_v.at[b], sem_l[b])

    def wait_load(j, b):
        pltpu.make_async_copy(msg_hbm.at[pl.ds(ebase + j * gch, gch)],
                              rows_v.at[b], sem_l[b]).wait()

    def drain_scatter(b):
        pltpu.make_async_copy(rows_v.at[b], acc.at[idx_v.at[0]],
                              sem_s[b]).wait()

    def visit(j, b, bp):
        wait_load(j, b)
        pltpu.async_copy(rows_v.at[b], acc.at[idx_v.at[j]], sem_s[b],
                         add=True)

        @pl.when(j >= 1)
        def _():
            drain_scatter(bp)

        @pl.when(j + 2 < n_chunks)
        def _():
            fire_load(j + 2, bp)

    fire_load(0, 0)
    fire_load(1, 1)

    @pl.loop(0, n_chunks // 3)
    def _(t):
        for b in range(3):
            visit(3 * t + b, b, (b + 2) % 3)

    rem = n_chunks % 3
    for i in range(rem):
        j = n_chunks - rem + i
        visit(j, j % 3, (j + 2) % 3)
    drain_scatter((n_chunks - 1) % 3)


@functools.cache
def _sc_scatter(nes, gch):
    k = len(nes)
    ews = tuple(ne // NW for ne in nes)
    nchs = tuple(ew // gch for ew in ews)

    @functools.partial(
        pl.kernel,
        out_type=jax.ShapeDtypeStruct((NC, N_PAD, D), jnp.float32),
        mesh=_sc_mesh(),
        scratch_types=(
            [pltpu.VMEM((nch, gch), jnp.int32) for nch in nchs]
            + [pltpu.VMEM((3, gch, D), jnp.float32),
               pltpu.VMEM_SHARED((N_PAD, D), jnp.float32)]
            + [pltpu.SemaphoreType.DMA] * 6
        ),
    )
    def body(*refs):
        msgs = refs[:k]
        dsts = refs[k:2 * k]
        zeros_hbm = refs[2 * k]
        part_hbm = refs[2 * k + 1]
        idxs = refs[2 * k + 2:3 * k + 2]
        rows_v = refs[3 * k + 2]
        acc = refs[3 * k + 3]
        sem_l = refs[3 * k + 4:3 * k + 7]
        sem_s = refs[3 * k + 7:3 * k + 10]

        c = lax.axis_index("c")
        s = lax.axis_index("s")
        wid = s * NC + c
        rbase = s * ROWS_PER_TILE

        pltpu.sync_copy(zeros_hbm.at[pl.ds(rbase, ROWS_PER_TILE)],
                        acc.at[pl.ds(rbase, ROWS_PER_TILE)])
        for i in range(k):
            pltpu.sync_copy(dsts[i].at[wid], idxs[i])
        plsc.subcore_barrier()

        for i in range(k):
            _scatter_ring(msgs[i], idxs[i], rows_v, sem_l, sem_s, acc,
                          wid * ews[i], gch, nchs[i])

        plsc.subcore_barrier()
        pltpu.sync_copy(acc.at[pl.ds(rbase, ROWS_PER_TILE)],
                        part_hbm.at[c, pl.ds(rbase, ROWS_PER_TILE)])

    return body


_NB = 2000
_EB = 2560

_full = lambda shape: pl.BlockSpec(shape, lambda i: (0,) * len(shape))


def _pre_body(x_ref, wa_ref, wb_ref, bin_ref, pa_ref, pb_ref):
    xb = x_ref[...]
    pa_ref[...] = jnp.dot(xb, wa_ref[...], preferred_element_type=jnp.float32)
    pb_ref[...] = (jnp.dot(xb, wb_ref[...], preferred_element_type=jnp.float32)
                   + bin_ref[...])


def _tc_pre(x, wa, wb, b_in):
    return pl.pallas_call(
        _pre_body,
        grid=(N_NODES // _NB,),
        in_specs=[
            pl.BlockSpec((_NB, D), lambda i: (i, 0)),
            _full((D, D)), _full((D, D)), _full((1, D)),
        ],
        out_specs=[
            pl.BlockSpec((_NB, D), lambda i: (i, 0)),
            pl.BlockSpec((_NB, D), lambda i: (i, 0)),
        ],
        out_shape=[
            jax.ShapeDtypeStruct((N_NODES, D), jnp.float32),
            jax.ShapeDtypeStruct((N_NODES, D), jnp.float32),
        ],
    )(x, wa, wb, b_in)


def _msg_body(g_ref, ea_ref, wc_ref, w1_ref, b1_ref, w2_ref, b2_ref,
              out_ref):
    bf = jnp.bfloat16
    h = g_ref[...] + jnp.dot(
        ea_ref[...], wc_ref[...], preferred_element_type=jnp.float32)
    h = jnp.maximum(h, 0.0)
    h = h + jnp.maximum(
        jnp.dot(h.astype(bf), w1_ref[...].astype(bf),
                preferred_element_type=jnp.float32)
        + b1_ref[...], 0.0)
    out_ref[...] = h + jnp.maximum(
        jnp.dot(h.astype(bf), w2_ref[...].astype(bf),
                preferred_element_type=jnp.float32)
        + b2_ref[...], 0.0)


def _tc_msg(g, ea, wc, w1, b1, w2, b2, lo):
    ne = g.shape[0]
    eb = next(e for e in range(_EB, 0, -8) if ne % e == 0 and lo % e == 0)
    off = lo // eb
    return pl.pallas_call(
        _msg_body,
        grid=(ne // eb,),
        in_specs=[
            pl.BlockSpec((eb, D), lambda i: (i, 0)),
            pl.BlockSpec((eb, D_EDGE), lambda i: (i + off, 0)),
            _full((D_EDGE, D)), _full((D, D)), _full((1, D)),
            _full((D, D)), _full((1, D)),
        ],
        out_specs=pl.BlockSpec((eb, D), lambda i: (i, 0)),
        out_shape=jax.ShapeDtypeStruct((ne, D), jnp.float32),
    )(g, ea, wc, w1, b1, w2, b2)


def _upd_body(p0_ref, p1_ref, x_ref, wi_ref, bi_ref,
              w1_ref, b1_ref, w2_ref, b2_ref, out_ref):
    agg = p0_ref[...] + p1_ref[...]
    h = jnp.maximum(
        jnp.dot(agg, wi_ref[...], preferred_element_type=jnp.float32)
        + bi_ref[...], 0.0)
    h = h + jnp.maximum(
        jnp.dot(h, w1_ref[...], preferred_element_type=jnp.float32)
        + b1_ref[...], 0.0)
    h = h + jnp.maximum(
        jnp.dot(h, w2_ref[...], preferred_element_type=jnp.float32)
        + b2_ref[...], 0.0)
    out_ref[...] = x_ref[...] + h


def _tc_upd(parts, x, wi, bi, w1, b1, w2, b2):
    nblk = pl.BlockSpec((_NB, D), lambda i: (i, 0))
    return pl.pallas_call(
        _upd_body,
        grid=(N_NODES // _NB,),
        in_specs=[
            nblk, nblk, nblk,
            _full((D, D)), _full((1, D)),
            _full((D, D)), _full((1, D)),
            _full((D, D)), _full((1, D)),
        ],
        out_specs=nblk,
        out_shape=jax.ShapeDtypeStruct((N_NODES, D), jnp.float32),
    )(*parts, x, wi, bi, w1, b1, w2, b2)


def kernel(x, edge_index, edge_attr,
           msg_Win, msg_bin, msg_W1, msg_b1, msg_W2, msg_b2,
           upd_Win, upd_bin, upd_W1, upd_b1, upd_W2, upd_b2):
    src = edge_index[0].astype(jnp.int32)
    dst = edge_index[1].astype(jnp.int32)
    wa = msg_Win[:D]
    wb = msg_Win[D:2 * D]
    wc = msg_Win[2 * D:]

    pa, pb = _tc_pre(x, wa, wb, msg_bin.reshape(1, D))
    zeros = jnp.zeros((N_PAD, D), jnp.float32)

    gch = 80
    sizes = (107520, 107520, 104960)
    ea16 = edge_attr.astype(jnp.bfloat16)
    wc16 = wc.astype(jnp.bfloat16)
    msgs, dst3s = [], []
    lo = 0
    for ne in sizes:
        g = _sc_gather(ne, gch, lo)(pa, pb, src, dst)
        msg = _tc_msg(g, ea16, wc16,
                      msg_W1, msg_b1.reshape(1, D),
                      msg_W2, msg_b2.reshape(1, D), lo)
        msgs.append(msg)
        dst3s.append(dst[lo:lo + ne].reshape(NW, ne // NW // gch, gch))
        lo += ne

    part = _sc_scatter(sizes, gch)(*msgs, *dst3s, zeros)
    parts = [part[0, :N_NODES], part[1, :N_NODES]]

    out = _tc_upd(parts, x,
                  upd_Win, upd_bin.reshape(1, D),
                  upd_W1, upd_b1.reshape(1, D), upd_W2, upd_b2.reshape(1, D))
    return out
